# Initial kernel scaffold; baseline (speedup 1.0000x reference)
import jax
import jax.numpy as jnp
from jax.experimental import pallas as pl


def kernel(z, edge_index):
    raise NotImplementedError("write your pallas kernel here")


# speedup vs baseline: 1.8457x; 1.8457x over previous
import functools

import jax
import jax.numpy as jnp
from jax.experimental import pallas as pl
from jax.experimental.pallas import tpu as pltpu

_TE = 2048
_U = 8


def _gather_dot_kernel(idx_ref, z_ref, o_ref, prod_ref, idx_s, sem, *, te, u):
    cp = pltpu.make_async_copy(idx_ref, idx_s, sem)
    cp.start()
    cp.wait()

    def chunk(ci, carry):
        base = pl.multiple_of(ci * u, u)
        rows = []
        for j in range(u):
            r = idx_s[0, 0, base + j]
            c = idx_s[0, 1, base + j]
            rows.append(z_ref[r] * z_ref[c])
        prod_ref[pl.ds(base, u), :] = jnp.concatenate(rows, axis=0)
        return carry

    jax.lax.fori_loop(0, te // u, chunk, 0)

    p = prod_ref[...]
    ones = jnp.ones((1, p.shape[1]), jnp.float32)
    s = jax.lax.dot_general(
        ones, p, dimension_numbers=(((1,), (1,)), ((), ())),
        precision=jax.lax.Precision.HIGHEST,
        preferred_element_type=jnp.float32)
    o_ref[...] = jax.nn.sigmoid(s)


@jax.jit
def kernel(z, edge_index):
    z = jnp.asarray(z, jnp.float32)
    n, d = z.shape
    row = jnp.asarray(edge_index[0], jnp.int32)
    col = jnp.asarray(edge_index[1], jnp.int32)
    e = int(row.shape[0])
    if e == 0:
        return jnp.zeros((0,), dtype=z.dtype)

    te = _TE
    nt = pl.cdiv(e, te)
    e_pad = nt * te
    row_p = jnp.pad(row, (0, e_pad - e))
    col_p = jnp.pad(col, (0, e_pad - e))
    idx = jnp.stack([row_p.reshape(nt, te), col_p.reshape(nt, te)], axis=1)
    z3 = z.reshape(n, 1, d)

    kern = functools.partial(_gather_dot_kernel, te=te, u=_U)
    out = pl.pallas_call(
        kern,
        out_shape=jax.ShapeDtypeStruct((1, e_pad), jnp.float32),
        grid=(nt,),
        in_specs=[
            pl.BlockSpec((1, 2, te), lambda i: (i, 0, 0)),
            pl.BlockSpec((n, 1, d), lambda i: (0, 0, 0)),
        ],
        out_specs=pl.BlockSpec((1, te), lambda i: (0, i)),
        scratch_shapes=[
            pltpu.VMEM((te, d), jnp.float32),
            pltpu.SMEM((1, 2, te), jnp.int32),
            pltpu.SemaphoreType.DMA,
        ],
        compiler_params=pltpu.CompilerParams(
            dimension_semantics=("parallel",),
            vmem_limit_bytes=48 * 1024 * 1024),
    )(idx, z3)
    return out[0, :e]


# speedup vs baseline: 3.6628x; 1.9845x over previous
import functools

import jax
import jax.numpy as jnp
from jax.experimental import pallas as pl
from jax.experimental.pallas import tpu as pltpu

_TE = 8192
_U = 16


def _gather_dot_kernel(row_ref, col_ref, z_ref, o_ref, prod_ref, row_s, col_s,
                       sem, *, te, u):
    cr = pltpu.make_async_copy(row_ref, row_s, sem.at[0])
    cc = pltpu.make_async_copy(col_ref, col_s, sem.at[1])
    cr.start()
    cc.start()
    cr.wait()
    cc.wait()

    def chunk(ci, carry):
        base = pl.multiple_of(ci * u, u)
        for j in range(u):
            r = row_s[0, 0, base + j]
            c = col_s[0, 0, base + j]
            prod_ref[pl.ds(base + j, 1), :] = z_ref[r] * z_ref[c]
        return carry

    jax.lax.fori_loop(0, te // u, chunk, 0)

    p = prod_ref[...]
    ones = jnp.ones((1, p.shape[1]), jnp.float32)
    s = jax.lax.dot_general(
        ones, p, dimension_numbers=(((1,), (1,)), ((), ())),
        precision=jax.lax.Precision.HIGHEST,
        preferred_element_type=jnp.float32)
    o_ref[...] = jax.nn.sigmoid(s)


@jax.jit
def kernel(z, edge_index):
    z = jnp.asarray(z, jnp.float32)
    n, d = z.shape
    row = jnp.asarray(edge_index[0], jnp.int32)
    col = jnp.asarray(edge_index[1], jnp.int32)
    e = int(row.shape[0])
    if e == 0:
        return jnp.zeros((0,), dtype=z.dtype)

    te = _TE
    nt = pl.cdiv(e, te)
    e_pad = nt * te
    row_p = jnp.pad(row, (0, e_pad - e)).reshape(nt, 1, te)
    col_p = jnp.pad(col, (0, e_pad - e)).reshape(nt, 1, te)
    z3 = z.reshape(n, 1, d)

    kern = functools.partial(_gather_dot_kernel, te=te, u=_U)
    out = pl.pallas_call(
        kern,
        out_shape=jax.ShapeDtypeStruct((1, e_pad), jnp.float32),
        grid=(nt,),
        in_specs=[
            pl.BlockSpec((1, 1, te), lambda i: (i, 0, 0)),
            pl.BlockSpec((1, 1, te), lambda i: (i, 0, 0)),
            pl.BlockSpec((n, 1, d), lambda i: (0, 0, 0)),
        ],
        out_specs=pl.BlockSpec((1, te), lambda i: (0, i)),
        scratch_shapes=[
            pltpu.VMEM((te, d), jnp.float32),
            pltpu.SMEM((1, 1, te), jnp.int32),
            pltpu.SMEM((1, 1, te), jnp.int32),
            pltpu.SemaphoreType.DMA((2,)),
        ],
        compiler_params=pltpu.CompilerParams(
            dimension_semantics=("parallel",),
            vmem_limit_bytes=48 * 1024 * 1024),
    )(row_p, col_p, z3)
    return out[0, :e]


# speedup vs baseline: 3.9033x; 1.0657x over previous
import functools

import jax
import jax.numpy as jnp
from jax.experimental import pallas as pl
from jax.experimental.pallas import tpu as pltpu

_TE = 8192
_U = 32


def _gather_dot_kernel(row_ref, col_ref, z_ref, o_ref, prod_ref, row_s, col_s,
                       sem, *, te, u):
    cr = pltpu.make_async_copy(row_ref, row_s, sem.at[0])
    cc = pltpu.make_async_copy(col_ref, col_s, sem.at[1])
    cr.start()
    cc.start()
    cr.wait()
    cc.wait()

    def chunk(ci, carry):
        base = pl.multiple_of(ci * u, u)
        for j in range(u):
            r = row_s[0, 0, base + j]
            c = col_s[0, 0, base + j]
            prod_ref[pl.ds(base + j, 1), :] = z_ref[r] * z_ref[c]
        return carry

    jax.lax.fori_loop(0, te // u, chunk, 0)

    p = prod_ref[...]
    ones = jnp.ones((1, p.shape[1]), jnp.float32)
    s = jax.lax.dot_general(
        ones, p, dimension_numbers=(((1,), (1,)), ((), ())),
        precision=jax.lax.Precision.HIGHEST,
        preferred_element_type=jnp.float32)
    o_ref[...] = jax.nn.sigmoid(s)


@jax.jit
def kernel(z, edge_index):
    z = jnp.asarray(z, jnp.float32)
    n, d = z.shape
    row = jnp.asarray(edge_index[0], jnp.int32)
    col = jnp.asarray(edge_index[1], jnp.int32)
    e = int(row.shape[0])
    if e == 0:
        return jnp.zeros((0,), dtype=z.dtype)

    te = _TE
    nt = pl.cdiv(e, te)
    e_pad = nt * te
    row_p = jnp.pad(row, (0, e_pad - e)).reshape(nt, 1, te)
    col_p = jnp.pad(col, (0, e_pad - e)).reshape(nt, 1, te)
    z3 = z.reshape(n, 1, d)

    kern = functools.partial(_gather_dot_kernel, te=te, u=_U)
    out = pl.pallas_call(
        kern,
        out_shape=jax.ShapeDtypeStruct((1, e_pad), jnp.float32),
        grid=(nt,),
        in_specs=[
            pl.BlockSpec((1, 1, te), lambda i: (i, 0, 0)),
            pl.BlockSpec((1, 1, te), lambda i: (i, 0, 0)),
            pl.BlockSpec((n, 1, d), lambda i: (0, 0, 0)),
        ],
        out_specs=pl.BlockSpec((1, te), lambda i: (0, i)),
        scratch_shapes=[
            pltpu.VMEM((te, d), jnp.float32),
            pltpu.SMEM((1, 1, te), jnp.int32),
            pltpu.SMEM((1, 1, te), jnp.int32),
            pltpu.SemaphoreType.DMA((2,)),
        ],
        compiler_params=pltpu.CompilerParams(
            dimension_semantics=("parallel",),
            vmem_limit_bytes=48 * 1024 * 1024),
    )(row_p, col_p, z3)
    return out[0, :e]


# speedup vs baseline: 4.1197x; 1.0554x over previous
import functools

import jax
import jax.numpy as jnp
from jax.experimental import pallas as pl
from jax.experimental.pallas import tpu as pltpu

_TE = 8192
_U = 32


def _gather_dot_kernel(row_ref, col_ref, z_ref, o_ref, prod_ref, row_s, col_s,
                       sem, *, te, u, nc):
    cs = te // nc
    copies = []
    for k in range(nc):
        sl = pl.ds(k * cs, cs)
        cr = pltpu.make_async_copy(row_ref.at[:, :, sl], row_s.at[:, :, sl],
                                   sem.at[0, k])
        cc = pltpu.make_async_copy(col_ref.at[:, :, sl], col_s.at[:, :, sl],
                                   sem.at[1, k])
        cr.start()
        cc.start()
        copies.append((cr, cc))

    def chunk(ci, carry):
        base = pl.multiple_of(ci * u, u)
        for j in range(u):
            r = row_s[0, 0, base + j]
            c = col_s[0, 0, base + j]
            prod_ref[pl.ds(base + j, 1), :] = z_ref[r] * z_ref[c]
        return carry

    it_per_chunk = cs // u
    for k in range(nc):
        cr, cc = copies[k]
        cr.wait()
        cc.wait()
        jax.lax.fori_loop(k * it_per_chunk, (k + 1) * it_per_chunk, chunk, 0)

    p = prod_ref[...]
    ones = jnp.ones((1, p.shape[1]), jnp.float32)
    s = jax.lax.dot_general(
        ones, p, dimension_numbers=(((1,), (1,)), ((), ())),
        precision=jax.lax.Precision.HIGHEST,
        preferred_element_type=jnp.float32)
    o_ref[...] = jax.nn.sigmoid(s)


@jax.jit
def kernel(z, edge_index):
    z = jnp.asarray(z, jnp.float32)
    n, d = z.shape
    row = jnp.asarray(edge_index[0], jnp.int32)
    col = jnp.asarray(edge_index[1], jnp.int32)
    e = int(row.shape[0])
    if e == 0:
        return jnp.zeros((0,), dtype=z.dtype)

    te = _TE
    nt = pl.cdiv(e, te)
    e_pad = nt * te
    row_p = jnp.pad(row, (0, e_pad - e)).reshape(nt, 1, te)
    col_p = jnp.pad(col, (0, e_pad - e)).reshape(nt, 1, te)
    z3 = z.reshape(n, 1, d)

    kern = functools.partial(_gather_dot_kernel, te=te, u=_U, nc=8)
    out = pl.pallas_call(
        kern,
        out_shape=jax.ShapeDtypeStruct((1, e_pad), jnp.float32),
        grid=(nt,),
        in_specs=[
            pl.BlockSpec((1, 1, te), lambda i: (i, 0, 0)),
            pl.BlockSpec((1, 1, te), lambda i: (i, 0, 0)),
            pl.BlockSpec((n, 1, d), lambda i: (0, 0, 0)),
        ],
        out_specs=pl.BlockSpec((1, te), lambda i: (0, i)),
        scratch_shapes=[
            pltpu.VMEM((te, d), jnp.float32),
            pltpu.SMEM((1, 1, te), jnp.int32),
            pltpu.SMEM((1, 1, te), jnp.int32),
            pltpu.SemaphoreType.DMA((2, 8)),
        ],
        compiler_params=pltpu.CompilerParams(
            dimension_semantics=("parallel",),
            vmem_limit_bytes=48 * 1024 * 1024),
    )(row_p, col_p, z3)
    return out[0, :e]


# speedup vs baseline: 4.7150x; 1.1445x over previous
import functools

import jax
import jax.numpy as jnp
from jax.experimental import pallas as pl
from jax.experimental.pallas import tpu as pltpu

_TE = 8192
_U = 32


def _gather_dot_kernel(row_ref, col_ref, z_ref, o_ref, prod_ref, row_s, col_s,
                       sem, *, te, u, nc):
    cs = te // nc
    copies = []
    for k in range(nc):
        sl = pl.ds(k * cs, cs)
        cr = pltpu.make_async_copy(row_ref.at[:, :, sl], row_s.at[:, :, sl],
                                   sem.at[0, k])
        cc = pltpu.make_async_copy(col_ref.at[:, :, sl], col_s.at[:, :, sl],
                                   sem.at[1, k])
        cr.start()
        cc.start()
        copies.append((cr, cc))

    def chunk(ci, carry):
        base = pl.multiple_of(ci * u, u)
        for j in range(u):
            r = row_s[0, 0, base + j]
            c = col_s[0, 0, base + j]
            prod_ref[pl.ds(base + j, 1), :] = z_ref[r] * z_ref[c]
        return carry

    it_per_chunk = cs // u
    for k in range(nc):
        cr, cc = copies[k]
        cr.wait()
        cc.wait()
        jax.lax.fori_loop(k * it_per_chunk, (k + 1) * it_per_chunk, chunk, 0)

    p = prod_ref[...]
    p_hi = p.astype(jnp.bfloat16)
    p_lo = (p - p_hi.astype(jnp.float32)).astype(jnp.bfloat16)
    ones = jnp.ones((1, p.shape[1]), jnp.bfloat16)
    dims = (((1,), (1,)), ((), ()))
    s = jax.lax.dot_general(ones, p_hi, dimension_numbers=dims,
                            preferred_element_type=jnp.float32)
    s = s + jax.lax.dot_general(ones, p_lo, dimension_numbers=dims,
                                preferred_element_type=jnp.float32)
    o_ref[...] = jax.nn.sigmoid(s)


@jax.jit
def kernel(z, edge_index):
    z = jnp.asarray(z, jnp.float32)
    n, d = z.shape
    row = jnp.asarray(edge_index[0], jnp.int32)
    col = jnp.asarray(edge_index[1], jnp.int32)
    e = int(row.shape[0])
    if e == 0:
        return jnp.zeros((0,), dtype=z.dtype)

    te = _TE
    nt = pl.cdiv(e, te)
    e_pad = nt * te
    row_p = jnp.pad(row, (0, e_pad - e)).reshape(nt, 1, te)
    col_p = jnp.pad(col, (0, e_pad - e)).reshape(nt, 1, te)
    z3 = z.reshape(n, 1, d)

    kern = functools.partial(_gather_dot_kernel, te=te, u=_U, nc=8)
    out = pl.pallas_call(
        kern,
        out_shape=jax.ShapeDtypeStruct((1, e_pad), jnp.float32),
        grid=(nt,),
        in_specs=[
            pl.BlockSpec((1, 1, te), lambda i: (i, 0, 0)),
            pl.BlockSpec((1, 1, te), lambda i: (i, 0, 0)),
            pl.BlockSpec((n, 1, d), lambda i: (0, 0, 0)),
        ],
        out_specs=pl.BlockSpec((1, te), lambda i: (0, i)),
        scratch_shapes=[
            pltpu.VMEM((te, d), jnp.float32),
            pltpu.SMEM((1, 1, te), jnp.int32),
            pltpu.SMEM((1, 1, te), jnp.int32),
            pltpu.SemaphoreType.DMA((2, 8)),
        ],
        compiler_params=pltpu.CompilerParams(
            dimension_semantics=("parallel",),
            vmem_limit_bytes=48 * 1024 * 1024),
    )(row_p, col_p, z3)
    return out[0, :e]


# speedup vs baseline: 4.8095x; 1.0200x over previous
import functools

import jax
import jax.numpy as jnp
from jax.experimental import pallas as pl
from jax.experimental.pallas import tpu as pltpu

_TE = 8192
_U = 32


def _gather_dot_kernel(row_ref, col_ref, z_ref, o_ref, prod_ref, row_s, col_s,
                       sem, *, te, u, nc):
    cs = te // nc
    copies = []
    for k in range(nc):
        sl = pl.ds(k * cs, cs)
        cr = pltpu.make_async_copy(row_ref.at[:, :, sl], row_s.at[:, :, sl],
                                   sem.at[0, k])
        cc = pltpu.make_async_copy(col_ref.at[:, :, sl], col_s.at[:, :, sl],
                                   sem.at[1, k])
        cr.start()
        cc.start()
        copies.append((cr, cc))

    def chunk(ci, carry):
        base = pl.multiple_of(ci * u, u)
        for j in range(u):
            r = row_s[0, 0, base + j]
            c = col_s[0, 0, base + j]
            prod_ref[pl.ds(base + j, 1), :] = z_ref[r] * z_ref[c]
        return carry

    it_per_chunk = cs // u
    for k in range(nc):
        cr, cc = copies[k]
        cr.wait()
        cc.wait()
        jax.lax.fori_loop(k * it_per_chunk, (k + 1) * it_per_chunk, chunk, 0)

    p = prod_ref[...]
    p_hi = p.astype(jnp.bfloat16)
    p_lo = (p - p_hi.astype(jnp.float32)).astype(jnp.bfloat16)
    ones = jnp.ones((1, p.shape[1]), jnp.bfloat16)
    dims = (((1,), (1,)), ((), ()))
    s = jax.lax.dot_general(ones, p_hi, dimension_numbers=dims,
                            preferred_element_type=jnp.float32)
    s = s + jax.lax.dot_general(ones, p_lo, dimension_numbers=dims,
                                preferred_element_type=jnp.float32)
    o_ref[...] = jax.nn.sigmoid(s).reshape(o_ref.shape)


@jax.jit
def kernel(z, edge_index):
    z = jnp.asarray(z, jnp.float32)
    n, d = z.shape
    row = jnp.asarray(edge_index[0], jnp.int32)
    col = jnp.asarray(edge_index[1], jnp.int32)
    e = int(row.shape[0])
    if e == 0:
        return jnp.zeros((0,), dtype=z.dtype)

    te = _TE
    nt = pl.cdiv(e, te)
    e_pad = nt * te
    row_p = jnp.pad(row, (0, e_pad - e)).reshape(nt, 1, te)
    col_p = jnp.pad(col, (0, e_pad - e)).reshape(nt, 1, te)
    z3 = z.reshape(n, 1, d)

    kern = functools.partial(_gather_dot_kernel, te=te, u=_U, nc=8)
    out = pl.pallas_call(
        kern,
        out_shape=jax.ShapeDtypeStruct((e_pad,), jnp.float32),
        grid=(nt,),
        in_specs=[
            pl.BlockSpec((1, 1, te), lambda i: (i, 0, 0)),
            pl.BlockSpec((1, 1, te), lambda i: (i, 0, 0)),
            pl.BlockSpec((n, 1, d), lambda i: (0, 0, 0)),
        ],
        out_specs=pl.BlockSpec((te,), lambda i: (i,)),
        scratch_shapes=[
            pltpu.VMEM((te, d), jnp.float32),
            pltpu.SMEM((1, 1, te), jnp.int32),
            pltpu.SMEM((1, 1, te), jnp.int32),
            pltpu.SemaphoreType.DMA((2, 8)),
        ],
        compiler_params=pltpu.CompilerParams(
            dimension_semantics=("parallel",),
            vmem_limit_bytes=48 * 1024 * 1024),
    )(row_p, col_p, z3)
    return out[:e]


# speedup vs baseline: 5.5155x; 1.1468x over previous
import functools

import jax
import jax.numpy as jnp
from jax.experimental import pallas as pl
from jax.experimental.pallas import tpu as pltpu

_TE = 8192
_MC = 128
_NC = 8


def _reduce_chunk(prod_ref, o_ref, c0, d):
    pc = prod_ref[pl.ds(c0, _MC), :]
    p_hi = pc.astype(jnp.bfloat16)
    p_lo = (pc - p_hi.astype(jnp.float32)).astype(jnp.bfloat16)
    ones = jnp.ones((1, d), jnp.bfloat16)
    dims = (((1,), (1,)), ((), ()))
    s = jax.lax.dot_general(ones, p_hi, dimension_numbers=dims,
                            preferred_element_type=jnp.float32)
    s = s + jax.lax.dot_general(ones, p_lo, dimension_numbers=dims,
                                preferred_element_type=jnp.float32)
    o_ref[pl.ds(c0, _MC)] = jax.nn.sigmoid(s).reshape(_MC)


def _gather_dot_kernel(row_ref, col_ref, z_ref, o_ref, prod_ref, row_s, col_s,
                       sem, *, te, nc):
    d = z_ref.shape[2]
    mc = _MC
    n_macro = te // mc
    cs = te // nc
    macro_per_stage = n_macro // nc

    copies = []
    for k in range(nc):
        sl = pl.ds(k * cs, cs)
        cr = pltpu.make_async_copy(row_ref.at[:, :, sl], row_s.at[:, :, sl],
                                   sem.at[0, k])
        cc = pltpu.make_async_copy(col_ref.at[:, :, sl], col_s.at[:, :, sl],
                                   sem.at[1, k])
        cr.start()
        cc.start()
        copies.append((cr, cc))

    def gather_chunk(base):
        for j in range(mc):
            r = row_s[0, 0, base + j]
            c = col_s[0, 0, base + j]
            prod_ref[pl.ds(base + j, 1), :] = z_ref[r] * z_ref[c]

    def body(ci, carry):
        _reduce_chunk(prod_ref, o_ref, pl.multiple_of((ci - 1) * mc, mc), d)
        gather_chunk(pl.multiple_of(ci * mc, mc))
        return carry

    cr, cc = copies[0]
    cr.wait()
    cc.wait()
    gather_chunk(0)
    jax.lax.fori_loop(1, macro_per_stage, body, 0)
    for k in range(1, nc):
        cr, cc = copies[k]
        cr.wait()
        cc.wait()
        jax.lax.fori_loop(k * macro_per_stage, (k + 1) * macro_per_stage,
                          body, 0)

    _reduce_chunk(prod_ref, o_ref, (n_macro - 1) * mc, d)


@jax.jit
def kernel(z, edge_index):
    z = jnp.asarray(z, jnp.float32)
    n, d = z.shape
    row = jnp.asarray(edge_index[0], jnp.int32)
    col = jnp.asarray(edge_index[1], jnp.int32)
    e = int(row.shape[0])
    if e == 0:
        return jnp.zeros((0,), dtype=z.dtype)

    te = _TE
    nt = pl.cdiv(e, te)
    e_pad = nt * te
    row_p = jnp.pad(row, (0, e_pad - e)).reshape(nt, 1, te)
    col_p = jnp.pad(col, (0, e_pad - e)).reshape(nt, 1, te)
    z3 = z.reshape(n, 1, d)

    kern = functools.partial(_gather_dot_kernel, te=te, nc=_NC)
    out = pl.pallas_call(
        kern,
        out_shape=jax.ShapeDtypeStruct((e_pad,), jnp.float32),
        grid=(nt,),
        in_specs=[
            pl.BlockSpec((1, 1, te), lambda i: (i, 0, 0)),
            pl.BlockSpec((1, 1, te), lambda i: (i, 0, 0)),
            pl.BlockSpec((n, 1, d), lambda i: (0, 0, 0)),
        ],
        out_specs=pl.BlockSpec((te,), lambda i: (i,)),
        scratch_shapes=[
            pltpu.VMEM((te, d), jnp.float32),
            pltpu.SMEM((1, 1, te), jnp.int32),
            pltpu.SMEM((1, 1, te), jnp.int32),
            pltpu.SemaphoreType.DMA((2, _NC)),
        ],
        compiler_params=pltpu.CompilerParams(
            dimension_semantics=("parallel",),
            vmem_limit_bytes=48 * 1024 * 1024),
    )(row_p, col_p, z3)
    return out[:e]


# speedup vs baseline: 5.5359x; 1.0037x over previous
import functools

import jax
import jax.numpy as jnp
from jax.experimental import pallas as pl
from jax.experimental.pallas import tpu as pltpu

_TE = 8192
_MC = 128
_NC = 8


def _reduce_chunk(prod_ref, o_ref, c0, d):
    pc = prod_ref[pl.ds(c0, _MC), :]
    ones = jnp.ones((1, d), jnp.float32)
    dims = (((1,), (1,)), ((), ()))
    s = jax.lax.dot_general(ones, pc, dimension_numbers=dims,
                            preferred_element_type=jnp.float32)
    o_ref[pl.ds(c0, _MC)] = jax.nn.sigmoid(s).reshape(_MC)


def _gather_dot_kernel(row_ref, col_ref, z_ref, o_ref, prod_ref, row_s, col_s,
                       sem, *, te, nc):
    d = z_ref.shape[2]
    mc = _MC
    n_macro = te // mc
    cs = te // nc
    macro_per_stage = n_macro // nc

    copies = []
    for k in range(nc):
        sl = pl.ds(k * cs, cs)
        cr = pltpu.make_async_copy(row_ref.at[:, :, sl], row_s.at[:, :, sl],
                                   sem.at[0, k])
        cc = pltpu.make_async_copy(col_ref.at[:, :, sl], col_s.at[:, :, sl],
                                   sem.at[1, k])
        cr.start()
        cc.start()
        copies.append((cr, cc))

    def gather_chunk(base):
        for j in range(mc):
            r = row_s[0, 0, base + j]
            c = col_s[0, 0, base + j]
            prod_ref[pl.ds(base + j, 1), :] = z_ref[r] * z_ref[c]

    def body(ci, carry):
        _reduce_chunk(prod_ref, o_ref, pl.multiple_of((ci - 1) * mc, mc), d)
        gather_chunk(pl.multiple_of(ci * mc, mc))
        return carry

    cr, cc = copies[0]
    cr.wait()
    cc.wait()
    gather_chunk(0)
    jax.lax.fori_loop(1, macro_per_stage, body, 0)
    for k in range(1, nc):
        cr, cc = copies[k]
        cr.wait()
        cc.wait()
        jax.lax.fori_loop(k * macro_per_stage, (k + 1) * macro_per_stage,
                          body, 0)

    _reduce_chunk(prod_ref, o_ref, (n_macro - 1) * mc, d)


@jax.jit
def kernel(z, edge_index):
    z = jnp.asarray(z, jnp.float32)
    n, d = z.shape
    row = jnp.asarray(edge_index[0], jnp.int32)
    col = jnp.asarray(edge_index[1], jnp.int32)
    e = int(row.shape[0])
    if e == 0:
        return jnp.zeros((0,), dtype=z.dtype)

    te = _TE
    nt = pl.cdiv(e, te)
    e_pad = nt * te
    row_p = jnp.pad(row, (0, e_pad - e)).reshape(nt, 1, te)
    col_p = jnp.pad(col, (0, e_pad - e)).reshape(nt, 1, te)
    z3 = z.reshape(n, 1, d)

    kern = functools.partial(_gather_dot_kernel, te=te, nc=_NC)
    out = pl.pallas_call(
        kern,
        out_shape=jax.ShapeDtypeStruct((e_pad,), jnp.float32),
        grid=(nt,),
        in_specs=[
            pl.BlockSpec((1, 1, te), lambda i: (i, 0, 0)),
            pl.BlockSpec((1, 1, te), lambda i: (i, 0, 0)),
            pl.BlockSpec((n, 1, d), lambda i: (0, 0, 0)),
        ],
        out_specs=pl.BlockSpec((te,), lambda i: (i,)),
        scratch_shapes=[
            pltpu.VMEM((te, d), jnp.float32),
            pltpu.SMEM((1, 1, te), jnp.int32),
            pltpu.SMEM((1, 1, te), jnp.int32),
            pltpu.SemaphoreType.DMA((2, _NC)),
        ],
        compiler_params=pltpu.CompilerParams(
            dimension_semantics=("parallel",),
            vmem_limit_bytes=48 * 1024 * 1024),
    )(row_p, col_p, z3)
    return out[:e]


# speedup vs baseline: 5.6440x; 1.0195x over previous
import functools

import jax
import jax.numpy as jnp
from jax.experimental import pallas as pl
from jax.experimental.pallas import tpu as pltpu

_TE = 16384
_MC = 128
_NC = 16


def _reduce_chunk(prod_ref, o_ref, c0, d):
    pc = prod_ref[pl.ds(c0, _MC), :]
    p_hi = pc.astype(jnp.bfloat16)
    p_lo = (pc - p_hi.astype(jnp.float32)).astype(jnp.bfloat16)
    ones = jnp.ones((1, d), jnp.bfloat16)
    dims = (((1,), (1,)), ((), ()))
    s = jax.lax.dot_general(ones, p_hi, dimension_numbers=dims,
                            preferred_element_type=jnp.float32)
    s = s + jax.lax.dot_general(ones, p_lo, dimension_numbers=dims,
                                preferred_element_type=jnp.float32)
    o_ref[pl.ds(c0, _MC)] = jax.nn.sigmoid(s).reshape(_MC)


def _gather_dot_kernel(row_ref, col_ref, z_ref, o_ref, prod_ref, row_s, col_s,
                       sem, *, te, nc):
    d = z_ref.shape[2]
    mc = _MC
    n_macro = te // mc
    cs = te // nc
    macro_per_stage = n_macro // nc

    copies = []
    for k in range(nc):
        sl = pl.ds(k * cs, cs)
        cr = pltpu.make_async_copy(row_ref.at[:, :, sl], row_s.at[:, :, sl],
                                   sem.at[0, k])
        cc = pltpu.make_async_copy(col_ref.at[:, :, sl], col_s.at[:, :, sl],
                                   sem.at[1, k])
        cr.start()
        cc.start()
        copies.append((cr, cc))

    def gather_chunk(base):
        for j in range(mc):
            r = row_s[0, 0, base + j]
            c = col_s[0, 0, base + j]
            prod_ref[pl.ds(base + j, 1), :] = z_ref[r] * z_ref[c]

    def body(ci, carry):
        _reduce_chunk(prod_ref, o_ref, pl.multiple_of((ci - 1) * mc, mc), d)
        gather_chunk(pl.multiple_of(ci * mc, mc))
        return carry

    cr, cc = copies[0]
    cr.wait()
    cc.wait()
    gather_chunk(0)
    jax.lax.fori_loop(1, macro_per_stage, body, 0)
    for k in range(1, nc):
        cr, cc = copies[k]
        cr.wait()
        cc.wait()
        jax.lax.fori_loop(k * macro_per_stage, (k + 1) * macro_per_stage,
                          body, 0)

    _reduce_chunk(prod_ref, o_ref, (n_macro - 1) * mc, d)


@jax.jit
def kernel(z, edge_index):
    z = jnp.asarray(z, jnp.float32)
    n, d = z.shape
    row = jnp.asarray(edge_index[0], jnp.int32)
    col = jnp.asarray(edge_index[1], jnp.int32)
    e = int(row.shape[0])
    if e == 0:
        return jnp.zeros((0,), dtype=z.dtype)

    te = _TE
    nt = pl.cdiv(e, te)
    e_pad = nt * te
    row_p = jnp.pad(row, (0, e_pad - e)).reshape(nt, 1, te)
    col_p = jnp.pad(col, (0, e_pad - e)).reshape(nt, 1, te)
    z3 = z.reshape(n, 1, d)

    kern = functools.partial(_gather_dot_kernel, te=te, nc=_NC)
    out = pl.pallas_call(
        kern,
        out_shape=jax.ShapeDtypeStruct((e_pad,), jnp.float32),
        grid=(nt,),
        in_specs=[
            pl.BlockSpec((1, 1, te), lambda i: (i, 0, 0)),
            pl.BlockSpec((1, 1, te), lambda i: (i, 0, 0)),
            pl.BlockSpec((n, 1, d), lambda i: (0, 0, 0)),
        ],
        out_specs=pl.BlockSpec((te,), lambda i: (i,)),
        scratch_shapes=[
            pltpu.VMEM((te, d), jnp.float32),
            pltpu.SMEM((1, 1, te), jnp.int32),
            pltpu.SMEM((1, 1, te), jnp.int32),
            pltpu.SemaphoreType.DMA((2, _NC)),
        ],
        compiler_params=pltpu.CompilerParams(
            dimension_semantics=("parallel",),
            vmem_limit_bytes=48 * 1024 * 1024),
    )(row_p, col_p, z3)
    return out[:e]


# speedup vs baseline: 5.6452x; 1.0002x over previous
import functools

import jax
import jax.numpy as jnp
from jax.experimental import pallas as pl
from jax.experimental.pallas import tpu as pltpu

_TE = 16384
_MC = 128
_NC = 8


def _reduce_chunk(prod_ref, o_ref, c0, d):
    pc = prod_ref[pl.ds(c0, _MC), :]
    p_hi = pc.astype(jnp.bfloat16)
    p_lo = (pc - p_hi.astype(jnp.float32)).astype(jnp.bfloat16)
    ones = jnp.ones((1, d), jnp.bfloat16)
    dims = (((1,), (1,)), ((), ()))
    s = jax.lax.dot_general(ones, p_hi, dimension_numbers=dims,
                            preferred_element_type=jnp.float32)
    s = s + jax.lax.dot_general(ones, p_lo, dimension_numbers=dims,
                                preferred_element_type=jnp.float32)
    o_ref[pl.ds(c0, _MC)] = jax.nn.sigmoid(s).reshape(_MC)


def _gather_dot_kernel(row_ref, col_ref, z_ref, o_ref, prod_ref, row_s, col_s,
                       sem, *, te, nc):
    d = z_ref.shape[2]
    mc = _MC
    n_macro = te // mc
    cs = te // nc
    macro_per_stage = n_macro // nc

    copies = []
    for k in range(nc):
        sl = pl.ds(k * cs, cs)
        cr = pltpu.make_async_copy(row_ref.at[:, :, sl], row_s.at[:, :, sl],
                                   sem.at[0, k])
        cc = pltpu.make_async_copy(col_ref.at[:, :, sl], col_s.at[:, :, sl],
                                   sem.at[1, k])
        cr.start()
        cc.start()
        copies.append((cr, cc))

    def gather_chunk(base):
        for j in range(mc):
            r = row_s[0, 0, base + j]
            c = col_s[0, 0, base + j]
            prod_ref[pl.ds(base + j, 1), :] = z_ref[r] * z_ref[c]

    def body(ci, carry):
        _reduce_chunk(prod_ref, o_ref, pl.multiple_of((ci - 1) * mc, mc), d)
        gather_chunk(pl.multiple_of(ci * mc, mc))
        return carry

    cr, cc = copies[0]
    cr.wait()
    cc.wait()
    gather_chunk(0)
    jax.lax.fori_loop(1, macro_per_stage, body, 0)
    for k in range(1, nc):
        cr, cc = copies[k]
        cr.wait()
        cc.wait()
        jax.lax.fori_loop(k * macro_per_stage, (k + 1) * macro_per_stage,
                          body, 0)

    _reduce_chunk(prod_ref, o_ref, (n_macro - 1) * mc, d)


@jax.jit
def kernel(z, edge_index):
    z = jnp.asarray(z, jnp.float32)
    n, d = z.shape
    row = jnp.asarray(edge_index[0], jnp.int32)
    col = jnp.asarray(edge_index[1], jnp.int32)
    e = int(row.shape[0])
    if e == 0:
        return jnp.zeros((0,), dtype=z.dtype)

    te = _TE
    nt = pl.cdiv(e, te)
    e_pad = nt * te
    row_p = jnp.pad(row, (0, e_pad - e)).reshape(nt, 1, te)
    col_p = jnp.pad(col, (0, e_pad - e)).reshape(nt, 1, te)
    z3 = z.reshape(n, 1, d)

    kern = functools.partial(_gather_dot_kernel, te=te, nc=_NC)
    out = pl.pallas_call(
        kern,
        out_shape=jax.ShapeDtypeStruct((e_pad,), jnp.float32),
        grid=(nt,),
        in_specs=[
            pl.BlockSpec((1, 1, te), lambda i: (i, 0, 0)),
            pl.BlockSpec((1, 1, te), lambda i: (i, 0, 0)),
            pl.BlockSpec((n, 1, d), lambda i: (0, 0, 0)),
        ],
        out_specs=pl.BlockSpec((te,), lambda i: (i,)),
        scratch_shapes=[
            pltpu.VMEM((te, d), jnp.float32),
            pltpu.SMEM((1, 1, te), jnp.int32),
            pltpu.SMEM((1, 1, te), jnp.int32),
            pltpu.SemaphoreType.DMA((2, _NC)),
        ],
        compiler_params=pltpu.CompilerParams(
            dimension_semantics=("parallel",),
            vmem_limit_bytes=48 * 1024 * 1024),
    )(row_p, col_p, z3)
    return out[:e]


# speedup vs baseline: 5.7553x; 1.0195x over previous
import functools

import jax
import jax.numpy as jnp
from jax.experimental import pallas as pl
from jax.experimental.pallas import tpu as pltpu

_TE = 32768
_MC = 128
_NC = 16


def _reduce_chunk(prod_ref, o_ref, c0, d):
    pc = prod_ref[pl.ds(c0, _MC), :]
    p_hi = pc.astype(jnp.bfloat16)
    p_lo = (pc - p_hi.astype(jnp.float32)).astype(jnp.bfloat16)
    ones = jnp.ones((1, d), jnp.bfloat16)
    dims = (((1,), (1,)), ((), ()))
    s = jax.lax.dot_general(ones, p_hi, dimension_numbers=dims,
                            preferred_element_type=jnp.float32)
    s = s + jax.lax.dot_general(ones, p_lo, dimension_numbers=dims,
                                preferred_element_type=jnp.float32)
    o_ref[pl.ds(c0, _MC)] = jax.nn.sigmoid(s).reshape(_MC)


def _gather_dot_kernel(row_ref, col_ref, z_ref, o_ref, prod_ref, row_s, col_s,
                       sem, *, te, nc):
    d = z_ref.shape[2]
    mc = _MC
    n_macro = te // mc
    cs = te // nc
    macro_per_stage = n_macro // nc

    copies = []
    for k in range(nc):
        sl = pl.ds(k * cs, cs)
        cr = pltpu.make_async_copy(row_ref.at[:, :, sl], row_s.at[:, :, sl],
                                   sem.at[0, k])
        cc = pltpu.make_async_copy(col_ref.at[:, :, sl], col_s.at[:, :, sl],
                                   sem.at[1, k])
        cr.start()
        cc.start()
        copies.append((cr, cc))

    def gather_chunk(base):
        for j in range(mc):
            r = row_s[0, 0, base + j]
            c = col_s[0, 0, base + j]
            prod_ref[pl.ds(base + j, 1), :] = z_ref[r] * z_ref[c]

    def body(ci, carry):
        _reduce_chunk(prod_ref, o_ref, pl.multiple_of((ci - 1) * mc, mc), d)
        gather_chunk(pl.multiple_of(ci * mc, mc))
        return carry

    cr, cc = copies[0]
    cr.wait()
    cc.wait()
    gather_chunk(0)
    jax.lax.fori_loop(1, macro_per_stage, body, 0)
    for k in range(1, nc):
        cr, cc = copies[k]
        cr.wait()
        cc.wait()
        jax.lax.fori_loop(k * macro_per_stage, (k + 1) * macro_per_stage,
                          body, 0)

    _reduce_chunk(prod_ref, o_ref, (n_macro - 1) * mc, d)


@jax.jit
def kernel(z, edge_index):
    z = jnp.asarray(z, jnp.float32)
    n, d = z.shape
    row = jnp.asarray(edge_index[0], jnp.int32)
    col = jnp.asarray(edge_index[1], jnp.int32)
    e = int(row.shape[0])
    if e == 0:
        return jnp.zeros((0,), dtype=z.dtype)

    te = _TE
    nt = pl.cdiv(e, te)
    e_pad = nt * te
    row_p = jnp.pad(row, (0, e_pad - e)).reshape(nt, 1, te)
    col_p = jnp.pad(col, (0, e_pad - e)).reshape(nt, 1, te)
    z3 = z.reshape(n, 1, d)

    kern = functools.partial(_gather_dot_kernel, te=te, nc=_NC)
    out = pl.pallas_call(
        kern,
        out_shape=jax.ShapeDtypeStruct((e_pad,), jnp.float32),
        grid=(nt,),
        in_specs=[
            pl.BlockSpec((1, 1, te), lambda i: (i, 0, 0)),
            pl.BlockSpec((1, 1, te), lambda i: (i, 0, 0)),
            pl.BlockSpec((n, 1, d), lambda i: (0, 0, 0)),
        ],
        out_specs=pl.BlockSpec((te,), lambda i: (i,)),
        scratch_shapes=[
            pltpu.VMEM((te, d), jnp.float32),
            pltpu.SMEM((1, 1, te), jnp.int32),
            pltpu.SMEM((1, 1, te), jnp.int32),
            pltpu.SemaphoreType.DMA((2, _NC)),
        ],
        compiler_params=pltpu.CompilerParams(
            dimension_semantics=("parallel",),
            vmem_limit_bytes=48 * 1024 * 1024),
    )(row_p, col_p, z3)
    return out[:e]


# speedup vs baseline: 5.7932x; 1.0066x over previous
import functools

import jax
import jax.numpy as jnp
from jax.experimental import pallas as pl
from jax.experimental.pallas import tpu as pltpu

_TE = 32768
_MC = 128


def _reduce_chunk(prod_ref, o_ref, c0, d):
    pc = prod_ref[pl.ds(c0, _MC), :]
    p_hi = pc.astype(jnp.bfloat16)
    p_lo = (pc - p_hi.astype(jnp.float32)).astype(jnp.bfloat16)
    ones = jnp.ones((1, d), jnp.bfloat16)
    dims = (((1,), (1,)), ((), ()))
    s = jax.lax.dot_general(ones, p_hi, dimension_numbers=dims,
                            preferred_element_type=jnp.float32)
    s = s + jax.lax.dot_general(ones, p_lo, dimension_numbers=dims,
                                preferred_element_type=jnp.float32)
    o_ref[pl.ds(c0, _MC)] = jax.nn.sigmoid(s).reshape(_MC)


def _stage_bounds(n_macro):
    bounds = []
    start, size = 0, 1
    while start < n_macro:
        sz = min(size, n_macro - start)
        bounds.append((start, sz))
        start += sz
        if len(bounds) > 1:
            size *= 2
    return bounds


def _gather_dot_kernel(row_ref, col_ref, z_ref, o_ref, prod_ref, row_s, col_s,
                       sem, *, te):
    d = z_ref.shape[2]
    mc = _MC
    n_macro = te // mc
    stages = _stage_bounds(n_macro)

    copies = []
    for k, (ms, msz) in enumerate(stages):
        sl = pl.ds(ms * mc, msz * mc)
        cr = pltpu.make_async_copy(row_ref.at[:, :, sl], row_s.at[:, :, sl],
                                   sem.at[0, k])
        cc = pltpu.make_async_copy(col_ref.at[:, :, sl], col_s.at[:, :, sl],
                                   sem.at[1, k])
        cr.start()
        cc.start()
        copies.append((cr, cc))

    def gather_chunk(base):
        for j in range(mc):
            r = row_s[0, 0, base + j]
            c = col_s[0, 0, base + j]
            prod_ref[pl.ds(base + j, 1), :] = z_ref[r] * z_ref[c]

    def body(ci, carry):
        _reduce_chunk(prod_ref, o_ref, pl.multiple_of((ci - 1) * mc, mc), d)
        gather_chunk(pl.multiple_of(ci * mc, mc))
        return carry

    cr, cc = copies[0]
    cr.wait()
    cc.wait()
    gather_chunk(0)
    for k, (ms, msz) in enumerate(stages):
        if k == 0:
            lo, hi = 1, msz
        else:
            cr, cc = copies[k]
            cr.wait()
            cc.wait()
            lo, hi = ms, ms + msz
        if lo >= hi:
            continue
        if hi - lo == 1:
            body(lo, 0)
        else:
            jax.lax.fori_loop(lo, hi, body, 0)

    _reduce_chunk(prod_ref, o_ref, (n_macro - 1) * mc, d)


@jax.jit
def kernel(z, edge_index):
    z = jnp.asarray(z, jnp.float32)
    n, d = z.shape
    row = jnp.asarray(edge_index[0], jnp.int32)
    col = jnp.asarray(edge_index[1], jnp.int32)
    e = int(row.shape[0])
    if e == 0:
        return jnp.zeros((0,), dtype=z.dtype)

    te = _TE
    nt = pl.cdiv(e, te)
    e_pad = nt * te
    row_p = jnp.pad(row, (0, e_pad - e)).reshape(nt, 1, te)
    col_p = jnp.pad(col, (0, e_pad - e)).reshape(nt, 1, te)
    z3 = z.reshape(n, 1, d)

    nstages = len(_stage_bounds(te // _MC))
    kern = functools.partial(_gather_dot_kernel, te=te)
    out = pl.pallas_call(
        kern,
        out_shape=jax.ShapeDtypeStruct((e_pad,), jnp.float32),
        grid=(nt,),
        in_specs=[
            pl.BlockSpec((1, 1, te), lambda i: (i, 0, 0)),
            pl.BlockSpec((1, 1, te), lambda i: (i, 0, 0)),
            pl.BlockSpec((n, 1, d), lambda i: (0, 0, 0)),
        ],
        out_specs=pl.BlockSpec((te,), lambda i: (i,)),
        scratch_shapes=[
            pltpu.VMEM((te, d), jnp.float32),
            pltpu.SMEM((1, 1, te), jnp.int32),
            pltpu.SMEM((1, 1, te), jnp.int32),
            pltpu.SemaphoreType.DMA((2, nstages)),
        ],
        compiler_params=pltpu.CompilerParams(
            dimension_semantics=("parallel",),
            vmem_limit_bytes=48 * 1024 * 1024),
    )(row_p, col_p, z3)
    return out[:e]
